# baseline (speedup 1.0000x reference)
import functools

import jax
import jax.numpy as jnp
from jax import lax
from jax.experimental import pallas as pl
from jax.experimental.pallas import tpu as pltpu
from jax.experimental.pallas import tpu_sc as plsc

L = 16
NBUF = 3
RG = 8
CW = 4096


def _rcumsum_body(n_steps, nvec_c, num_cores, x_hbm, out_hbm,
                  b0, b1, b2, ls0, ls1, ls2, ss0, ss1, ss2):
    bufs = (b0, b1, b2)
    lsems = (ls0, ls1, ls2)
    ssems = (ss0, ss1, ss2)
    chunks_per_group = n_steps // 4
    wid = lax.axis_index("s") * num_cores + lax.axis_index("c")
    row0 = wid * (4 * RG)

    dnums = lax.GatherDimensionNumbers(
        offset_dims=(), collapsed_slice_dims=(0,), start_index_map=(0,))
    idx_last = jnp.full((L,), L - 1, dtype=jnp.int32)

    def slab(s):
        rows = pl.multiple_of(row0 + (s // chunks_per_group) * RG, RG)
        col0 = pl.multiple_of(
            (chunks_per_group - 1 - s % chunks_per_group) * CW, CW)
        return rows, col0

    def load(s, b):
        rows, col0 = slab(s)
        pltpu.make_async_copy(
            x_hbm.at[pl.ds(rows, RG), pl.ds(col0, CW)], bufs[b],
            lsems[b]).start()

    def wait_load(b):
        pltpu.make_async_copy(
            x_hbm.at[pl.ds(row0, RG), pl.ds(0, CW)], bufs[b],
            lsems[b]).wait()

    def store(s, b):
        rows, col0 = slab(s)
        pltpu.make_async_copy(
            bufs[b], out_hbm.at[pl.ds(rows, RG), pl.ds(col0, CW)],
            ssems[b]).start()

    def wait_store(b):
        pltpu.make_async_copy(
            bufs[b], out_hbm.at[pl.ds(row0, RG), pl.ds(0, CW)],
            ssems[b]).wait()

    def compute(buf, carries):
        out = []
        for r in range(RG):
            def step(i, carry, r=r):
                j = nvec_c - 1 - i
                base = pl.multiple_of(j * L, L)
                v = buf[r, pl.ds(base, L)]
                p = plsc.cumsum(v)
                tot = lax.gather(
                    p, idx_last[:, None], dnums, (1,),
                    mode=lax.GatherScatterMode.PROMISE_IN_BOUNDS)
                buf[r, pl.ds(base, L)] = carry + tot - p + v
                return carry + tot

            out.append(
                lax.fori_loop(0, nvec_c, step, carries[r], unroll=16))
        return tuple(out)

    load(0, 0)

    zero = jnp.zeros((L,), jnp.float32)
    init = (zero,) * RG

    def one_step(s, b, carries, has_next, need_drain):
        wait_load(b)
        nb = (b + 1) % NBUF
        if has_next:

            @pl.when(s + 1 < n_steps)
            def _():
                if need_drain:

                    @pl.when(s + 1 - NBUF >= 0)
                    def _():
                        wait_store(nb)

                load(s + 1, nb)

        fresh = s % chunks_per_group == 0
        carries = tuple(
            jnp.where(fresh, jnp.zeros((L,), jnp.float32), c)
            for c in carries)
        carries = compute(bufs[b], carries)
        store(s, b)
        return carries

    def outer(g, carries):
        for b in range(NBUF):
            carries = one_step(g * NBUF + b, b, carries,
                               has_next=True, need_drain=True)
        return carries

    n_full = (n_steps // NBUF) * NBUF
    carries = lax.fori_loop(0, n_full // NBUF, outer, init)
    for s in range(n_full, n_steps):
        carries = one_step(jnp.int32(s), s % NBUF, carries,
                           has_next=s + 1 < n_steps,
                           need_drain=s + 1 - NBUF >= 0)
    for b in range(NBUF):
        last_s = n_steps - NBUF + b
        if last_s >= 0:
            wait_store(last_s % NBUF)


def kernel(x):
    n_rows, n_cols = x.shape
    try:
        info = plsc.get_sparse_core_info()
        num_cores, num_subcores = info.num_cores, info.num_subcores
    except Exception:
        num_cores, num_subcores = 2, 16
    n_workers = num_cores * num_subcores
    assert n_rows % (n_workers * 4 * RG) == 0
    assert n_cols % CW == 0
    n_steps = (n_rows // (n_workers * RG)) * (n_cols // CW)

    mesh = plsc.VectorSubcoreMesh(
        core_axis_name="c", subcore_axis_name="s",
        num_cores=num_cores, num_subcores=num_subcores,
    )
    body = functools.partial(_rcumsum_body, n_steps, CW // L, num_cores)
    f = pl.kernel(
        body,
        out_type=jax.ShapeDtypeStruct((n_rows, n_cols), jnp.float32),
        mesh=mesh,
        scratch_types=(
            [pltpu.VMEM((RG, CW), jnp.float32)] * NBUF
            + [pltpu.SemaphoreType.DMA] * (2 * NBUF)
        ),
        compiler_params=pltpu.CompilerParams(needs_layout_passes=False),
    )
    return f(x)


# speedup vs baseline: 1.0771x; 1.0771x over previous
import functools

import jax
import jax.numpy as jnp
from jax import lax
from jax.experimental import pallas as pl
from jax.experimental.pallas import tpu as pltpu
from jax.experimental.pallas import tpu_sc as plsc

L = 16
NBUF = 6
PRE = 3


def _rcumsum_body(n_steps, nvec_c, num_cores, x_hbm, out_hbm, *scratch):
    bufs = scratch[:NBUF]
    lsems = scratch[NBUF:2 * NBUF]
    ssems = scratch[2 * NBUF:3 * NBUF]
    half = nvec_c * L
    wid = lax.axis_index("s") * num_cores + lax.axis_index("c")
    row0 = wid * (n_steps // 2)

    dnums = lax.GatherDimensionNumbers(
        offset_dims=(), collapsed_slice_dims=(0,), start_index_map=(0,))
    idx_last = jnp.full((L,), L - 1, dtype=jnp.int32)

    def chunk_slice(s):
        row = row0 + s // 2
        col0 = pl.multiple_of((1 - s % 2) * half, half)
        return row, col0

    def load(s, b):
        row, col0 = chunk_slice(s)
        pltpu.make_async_copy(
            x_hbm.at[row, pl.ds(col0, half)], bufs[b], lsems[b]).start()

    def wait_load(b):
        pltpu.make_async_copy(
            x_hbm.at[row0, pl.ds(0, half)], bufs[b], lsems[b]).wait()

    def store(s, b):
        row, col0 = chunk_slice(s)
        pltpu.make_async_copy(
            bufs[b], out_hbm.at[row, pl.ds(col0, half)], ssems[b]).start()

    def wait_store(b):
        pltpu.make_async_copy(
            bufs[b], out_hbm.at[row0, pl.ds(0, half)], ssems[b]).wait()

    def compute(buf, carry):
        def step(i, c):
            j = nvec_c - 1 - i
            base = pl.multiple_of(j * L, L)
            v = buf[pl.ds(base, L)]
            p = plsc.cumsum(v)
            tot = lax.gather(
                p, idx_last[:, None], dnums, (1,),
                mode=lax.GatherScatterMode.PROMISE_IN_BOUNDS)
            buf[pl.ds(base, L)] = c + tot - p + v
            return c + tot

        return lax.fori_loop(0, nvec_c, step, carry, unroll=16)

    def one_step(s, b, carry, has_next, need_drain):
        wait_load(b)
        nb = (b + PRE) % NBUF
        if has_next:

            @pl.when(s + PRE < n_steps)
            def _():
                if need_drain:

                    @pl.when(s + PRE - NBUF >= 0)
                    def _():
                        wait_store(nb)

                load(s + PRE, nb)

        carry = jnp.where(s % 2 == 0,
                          jnp.zeros((L,), jnp.float32), carry)
        carry = compute(bufs[b], carry)
        store(s, b)
        return carry

    for s in range(PRE):
        load(s, s)

    def outer(g, carry):
        for b in range(NBUF):
            carry = one_step(g * NBUF + b, b, carry,
                             has_next=True, need_drain=True)
        return carry

    n_full = (n_steps // NBUF) * NBUF
    carry = lax.fori_loop(0, n_full // NBUF, outer,
                          jnp.zeros((L,), jnp.float32))
    for s in range(n_full, n_steps):
        carry = one_step(jnp.int32(s), s % NBUF, carry,
                         has_next=s + PRE < n_steps,
                         need_drain=s + PRE - NBUF >= 0)
    for s in range(max(0, n_steps - NBUF), n_steps):
        wait_store(s % NBUF)


def kernel(x):
    n_rows, n_cols = x.shape
    try:
        info = plsc.get_sparse_core_info()
        num_cores, num_subcores = info.num_cores, info.num_subcores
    except Exception:
        num_cores, num_subcores = 2, 16
    n_workers = num_cores * num_subcores
    assert n_rows % n_workers == 0 and n_cols % (2 * L) == 0
    n_steps = (n_rows // n_workers) * 2
    nvec_c = n_cols // 2 // L

    mesh = plsc.VectorSubcoreMesh(
        core_axis_name="c", subcore_axis_name="s",
        num_cores=num_cores, num_subcores=num_subcores,
    )
    body = functools.partial(_rcumsum_body, n_steps, nvec_c, num_cores)
    f = pl.kernel(
        body,
        out_type=jax.ShapeDtypeStruct((n_rows, n_cols), jnp.float32),
        mesh=mesh,
        scratch_types=(
            [pltpu.VMEM((n_cols // 2,), jnp.float32)] * NBUF
            + [pltpu.SemaphoreType.DMA] * (2 * NBUF)
        ),
        compiler_params=pltpu.CompilerParams(needs_layout_passes=False),
    )
    return f(x)


# speedup vs baseline: 1.0931x; 1.0148x over previous
import functools

import jax
import jax.numpy as jnp
from jax import lax
from jax.experimental import pallas as pl
from jax.experimental.pallas import tpu as pltpu
from jax.experimental.pallas import tpu_sc as plsc

L = 16
NBUF = 3


def _rcumsum_body(nrows_per_worker, nvec, num_cores, x_hbm, out_hbm,
                  b0, b1, b2, ls0, ls1, ls2, ss0, ss1, ss2):
    bufs = (b0, b1, b2)
    lsems = (ls0, ls1, ls2)
    ssems = (ss0, ss1, ss2)
    n_cols = nvec * L
    wid = lax.axis_index("s") * num_cores + lax.axis_index("c")
    row0 = wid * nrows_per_worker

    dnums = lax.GatherDimensionNumbers(
        offset_dims=(), collapsed_slice_dims=(0,), start_index_map=(0,))
    idx_last = jnp.full((L,), L - 1, dtype=jnp.int32)

    def load(r, b):
        pltpu.make_async_copy(x_hbm.at[row0 + r], bufs[b], lsems[b]).start()

    def wait_load(b):
        pltpu.make_async_copy(x_hbm.at[row0], bufs[b], lsems[b]).wait()

    def store(r, b):
        pltpu.make_async_copy(bufs[b], out_hbm.at[row0 + r], ssems[b]).start()

    def wait_store(b):
        pltpu.make_async_copy(bufs[b], out_hbm.at[row0], ssems[b]).wait()

    def compute(buf):
        def step(i, carry):
            j = nvec - 1 - i
            base = pl.multiple_of(j * L, L)
            v = buf[pl.ds(base, L)]
            p = plsc.cumsum(v)
            tot = lax.gather(
                p, idx_last[:, None], dnums, (1,),
                mode=lax.GatherScatterMode.PROMISE_IN_BOUNDS)
            buf[pl.ds(base, L)] = carry + tot - p + v
            return carry + tot

        lax.fori_loop(0, nvec, step, jnp.zeros((L,), jnp.float32), unroll=16)

    load(0, 0)

    def outer(g, _):
        for b in range(NBUF):
            r = g * NBUF + b

            @pl.when(r < nrows_per_worker)
            def _():
                wait_load(b)
                nb = (b + 1) % NBUF

                @pl.when(r + 1 < nrows_per_worker)
                def _():
                    @pl.when(r + 1 - NBUF >= 0)
                    def _():
                        wait_store(nb)

                    load(r + 1, nb)

                compute(bufs[b])
                store(r, b)

        return 0

    n_outer = (nrows_per_worker + NBUF - 1) // NBUF
    lax.fori_loop(0, n_outer, outer, 0)
    for b in range(NBUF):
        last_r = nrows_per_worker - NBUF + b
        if last_r >= 0:
            wait_store((last_r) % NBUF)


def kernel(x):
    n_rows, n_cols = x.shape
    try:
        info = plsc.get_sparse_core_info()
        num_cores, num_subcores = info.num_cores, info.num_subcores
    except Exception:
        num_cores, num_subcores = 2, 16
    n_workers = num_cores * num_subcores
    assert n_rows % n_workers == 0 and n_cols % L == 0
    nrows_per_worker = n_rows // n_workers
    nvec = n_cols // L

    mesh = plsc.VectorSubcoreMesh(
        core_axis_name="c", subcore_axis_name="s",
        num_cores=num_cores, num_subcores=num_subcores,
    )
    body = functools.partial(_rcumsum_body, nrows_per_worker, nvec, num_cores)
    f = pl.kernel(
        body,
        out_type=jax.ShapeDtypeStruct((n_rows, n_cols), jnp.float32),
        mesh=mesh,
        scratch_types=(
            [pltpu.VMEM((n_cols,), jnp.float32)] * NBUF
            + [pltpu.SemaphoreType.DMA] * (2 * NBUF)
        ),
        compiler_params=pltpu.CompilerParams(needs_layout_passes=False),
    )
    return f(x)
